# baseline (speedup 1.0000x reference)
import functools

import jax
import jax.numpy as jnp
from jax import lax
from jax.experimental import pallas as pl
from jax.experimental.pallas import tpu as pltpu
from jax.experimental.pallas import tpu_sc as plsc

_HID = 64
_HEADS = 4
_HD = _HID // _HEADS
_G = 64
_N = 4096
_E = 65536

_NC = 2
_NS = 16
_EPW = _E // _NS
_CH = 128
_NCHUNK = _EPW // _CH
_RPS = _N // _NS


def _gine_agg(x_m, ea_m, src_m, dst_m, x_p, ea_p, src_p, dst_p):
    mesh = plsc.VectorSubcoreMesh(
        core_axis_name="c", subcore_axis_name="s",
        num_cores=_NC, num_subcores=_NS)

    @functools.partial(
        pl.kernel,
        mesh=mesh,
        compiler_params=pltpu.CompilerParams(use_tc_tiling_on_sc=False),
        out_type=jax.ShapeDtypeStruct((_NC, _N, _HID), jnp.float32),
        scratch_types=[
            pltpu.VMEM((4, _CH), jnp.int32),
            pltpu.VMEM((4, _CH), jnp.int32),
            pltpu.VMEM((4, _CH, _HID), jnp.float32),
            pltpu.VMEM((4, _CH, _HID), jnp.float32),
            pltpu.VMEM_SHARED((_N, _HID), jnp.float32),
            pltpu.SemaphoreType.DMA,
            pltpu.SemaphoreType.DMA,
            pltpu.SemaphoreType.DMA,
            pltpu.SemaphoreType.DMA,
            pltpu.SemaphoreType.DMA,
            pltpu.SemaphoreType.DMA,
            pltpu.SemaphoreType.DMA,
            pltpu.SemaphoreType.DMA,
        ],
    )
    def k(xm_hbm, eam_hbm, srcm_hbm, dstm_hbm, xp_hbm, eap_hbm, srcp_hbm,
          dstp_hbm, out_hbm, sidx, didx, rows, eabuf, acc, sem_in0, sem_in1,
          sem_in2, sem_in3, sem_g0, sem_g1, sem_g2, sem_g3):
        c = lax.axis_index("c")
        s = lax.axis_index("s")
        sem_in = (sem_in0, sem_in1, sem_in2, sem_in3)
        sem_g = (sem_g0, sem_g1, sem_g2, sem_g3)

        def zero_fill():
            @plsc.parallel_loop(0, _CH, step=1, unroll=8)
            def zero_row(i):
                for j in range(_HID // 16):
                    rows[0, i, pl.ds(j * 16, 16)] = jnp.zeros((16,),
                                                              jnp.float32)
            for r in range(_RPS // _CH):
                pltpu.sync_copy(rows.at[0],
                                acc.at[pl.ds(s * _RPS + r * _CH, _CH)])
            plsc.subcore_barrier()

        def run_graph(x_hbm, ea_hbm, src_hbm, dst_hbm, gidx, zero_fill):
            base = s * _EPW

            def start_in(t, b):
                e0 = base + t * _CH
                pltpu.async_copy(src_hbm.at[pl.ds(e0, _CH)], sidx.at[b],
                                 sem_in[b])
                pltpu.async_copy(dst_hbm.at[pl.ds(e0, _CH)], didx.at[b],
                                 sem_in[b])
                pltpu.async_copy(ea_hbm.at[pl.ds(e0, _CH)], eabuf.at[b],
                                 sem_in[b])

            def wait_in(b):
                pltpu.make_async_copy(src_hbm.at[pl.ds(0, _CH)], sidx.at[b],
                                      sem_in[b]).wait()
                pltpu.make_async_copy(dst_hbm.at[pl.ds(0, _CH)], didx.at[b],
                                      sem_in[b]).wait()
                pltpu.make_async_copy(ea_hbm.at[pl.ds(0, _CH)], eabuf.at[b],
                                      sem_in[b]).wait()

            def start_gather(b):
                pltpu.async_copy(x_hbm.at[sidx.at[b]], rows.at[b], sem_g[b])

            def wait_gather(b):
                pltpu.make_async_copy(x_hbm.at[sidx.at[b]], rows.at[b],
                                      sem_g[b]).wait()

            def compute(b):
                @plsc.parallel_loop(0, _CH, step=1, unroll=8)
                def body(i):
                    for j in range(_HID // 16):
                        sl = pl.ds(j * 16, 16)
                        rows[b, i, sl] = jnp.maximum(
                            rows[b, i, sl] + eabuf[b, i, sl], 0.0)

            def scatter(b):
                pltpu.sync_copy(rows.at[b], acc.at[didx.at[b]], add=True)

            for b in range(4):
                start_in(b, b)
            zero_fill()
            wait_in(0)
            start_gather(0)
            wait_in(1)
            start_gather(1)

            def step(t, b):
                b2 = (b + 2) % 4
                @pl.when(t + 2 < _NCHUNK)
                def _():
                    wait_in(b2)
                    start_gather(b2)
                wait_gather(b)
                compute(b)
                scatter(b)

                @pl.when(t + 4 < _NCHUNK)
                def _():
                    start_in(t + 4, b)

            def quad(p, carry):
                for b in range(4):
                    step(4 * p + b, b)
                return carry
            lax.fori_loop(0, _NCHUNK // 4, quad, 0)

            plsc.subcore_barrier()
            pltpu.sync_copy(acc.at[pl.ds(s * _RPS, _RPS)],
                            out_hbm.at[gidx, pl.ds(s * _RPS, _RPS)])

        @pl.when(c == 0)
        def _():
            run_graph(xm_hbm, eam_hbm, srcm_hbm, dstm_hbm, 0, zero_fill)

        @pl.when(c == 1)
        def _():
            run_graph(xp_hbm, eap_hbm, srcp_hbm, dstp_hbm, 1, zero_fill)

    return k(x_m, ea_m, src_m, dst_m, x_p, ea_p, src_p, dst_p)


def _node_enc(x, W, b):
    n, f = x.shape

    def body(x_ref, w_ref, b_ref, o_ref):
        o_ref[...] = jnp.dot(x_ref[...], w_ref[...],
                             preferred_element_type=jnp.float32) + b_ref[...]

    return pl.pallas_call(
        body,
        out_shape=jax.ShapeDtypeStruct((n, _HID), jnp.float32),
    )(x, W, b.reshape(1, _HID))


def _edge_enc(ea, W, b):
    e, f = ea.shape
    be = 16384

    def body(ea_ref, w_ref, b_ref, o_ref):
        o_ref[...] = jnp.dot(ea_ref[...], w_ref[...],
                             preferred_element_type=jnp.float32) + b_ref[...]

    return pl.pallas_call(
        body,
        grid=(e // be,),
        in_specs=[pl.BlockSpec((be, f), lambda i: (i, 0)),
                  pl.BlockSpec((f, _HID), lambda i: (0, 0)),
                  pl.BlockSpec((1, _HID), lambda i: (0, 0))],
        out_specs=pl.BlockSpec((be, _HID), lambda i: (i, 0)),
        out_shape=jax.ShapeDtypeStruct((e, _HID), jnp.float32),
    )(ea, W, b.reshape(1, _HID))


def _gine_mlp(x, aggs, g, W1, b1, W2, b2):
    def body(x_ref, a_ref, w1_ref, b1_ref, w2_ref, b2_ref, o_ref):
        h = x_ref[...] + a_ref[0]
        h1 = jnp.maximum(
            jnp.dot(h, w1_ref[...], preferred_element_type=jnp.float32)
            + b1_ref[...], 0.0)
        h2 = (jnp.dot(h1, w2_ref[...], preferred_element_type=jnp.float32)
              + b2_ref[...])
        o_ref[...] = jnp.maximum(h2, 0.0)

    return pl.pallas_call(
        body,
        grid=(1,),
        in_specs=[pl.BlockSpec((_N, _HID), lambda i: (0, 0)),
                  pl.BlockSpec((1, _N, _HID), lambda i, _g=g: (_g, 0, 0)),
                  pl.BlockSpec((_HID, _HID), lambda i: (0, 0)),
                  pl.BlockSpec((1, _HID), lambda i: (0, 0)),
                  pl.BlockSpec((_HID, _HID), lambda i: (0, 0)),
                  pl.BlockSpec((1, _HID), lambda i: (0, 0))],
        out_specs=pl.BlockSpec((_N, _HID), lambda i: (0, 0)),
        out_shape=jax.ShapeDtypeStruct((_N, _HID), jnp.float32),
    )(x, aggs, W1, b1.reshape(1, _HID), W2, b2.reshape(1, _HID))


def _attn_prep(qn, kn, WQ, bQ, WK, bK, WV, bV):
    def body(qn_ref, kn_ref, wq, bq, wk, bk, wv, bv, q_out, kbd_out, vbd_out):
        q = (jnp.dot(qn_ref[...], wq[...], preferred_element_type=jnp.float32)
             + bq[...])
        q_out[...] = (q * (1.0 / (_HD ** 0.5))).astype(jnp.bfloat16)
        k = (jnp.dot(kn_ref[...], wk[...], preferred_element_type=jnp.float32)
             + bk[...])
        kt = k.T
        drow = lax.broadcasted_iota(jnp.int32, (_HID, 1), 0) // _HD
        for h in range(_HEADS):
            kbd_out[:, h * _N:(h + 1) * _N] = (
                kt * (drow == h).astype(jnp.float32)).astype(jnp.bfloat16)
        v = (jnp.dot(kn_ref[...], wv[...], preferred_element_type=jnp.float32)
             + bv[...])
        dcol = lax.broadcasted_iota(jnp.int32, (1, _HID), 1) // _HD
        hcol = lax.broadcasted_iota(jnp.int32, (1, _HEADS), 1)
        for h in range(_HEADS):
            vbd_out[h * _N:(h + 1) * _N, :_HID] = (
                v * (dcol == h).astype(jnp.float32)).astype(jnp.bfloat16)
            vbd_out[h * _N:(h + 1) * _N, _HID:] = jnp.broadcast_to(
                (hcol == h).astype(jnp.bfloat16), (_N, _HEADS))

    return pl.pallas_call(
        body,
        out_shape=(jax.ShapeDtypeStruct((_N, _HID), jnp.bfloat16),
                   jax.ShapeDtypeStruct((_HID, _HEADS * _N), jnp.bfloat16),
                   jax.ShapeDtypeStruct((_HEADS * _N, _HID + _HEADS),
                                        jnp.bfloat16)),
    )(qn, kn, WQ, bQ.reshape(1, _HID), WK, bK.reshape(1, _HID),
      WV, bV.reshape(1, _HID))


_QB = 512


def _attn_apply(q_scaled, k_bd, v_bd, qn):
    def body(q_ref, kbd_ref, vbd_ref, qn_ref, o_ref):
        s = jnp.dot(q_ref[...], kbd_ref[...],
                    preferred_element_type=jnp.float32)
        ws = []
        for h in range(_HEADS):
            sh = s[:, h * _N:(h + 1) * _N]
            m = jnp.max(sh, axis=1, keepdims=True)
            ws.append(jnp.exp(sh - m).astype(jnp.bfloat16))
        w = jnp.concatenate(ws, axis=1)
        acc = jnp.dot(w, vbd_ref[...], preferred_element_type=jnp.float32)
        o = acc[:, :_HID]
        recip = 1.0 / acc[:, _HID:]
        rec64 = jnp.concatenate(
            [jnp.broadcast_to(recip[:, h:h + 1], (_QB, _HD))
             for h in range(_HEADS)], axis=1)
        o_ref[...] = qn_ref[...] + o * rec64

    return pl.pallas_call(
        body,
        grid=(_N // _QB,),
        in_specs=[pl.BlockSpec((_QB, _HID), lambda i: (i, 0)),
                  pl.BlockSpec((_HID, _HEADS * _N), lambda i: (0, 0)),
                  pl.BlockSpec((_HEADS * _N, _HID + _HEADS), lambda i: (0, 0)),
                  pl.BlockSpec((_QB, _HID), lambda i: (i, 0))],
        out_specs=pl.BlockSpec((_QB, _HID), lambda i: (i, 0)),
        out_shape=jax.ShapeDtypeStruct((_N, _HID), jnp.float32),
    )(q_scaled, k_bd, v_bd, qn)


def _pool_head(hm, hp, bm, bp, fc1_W, fc1_b, fc2_W, fc2_b):
    def body(hm_ref, hp_ref, bm_ref, bp_ref, w1, b1, w2, b2, o_ref):
        gi = lax.broadcasted_iota(jnp.int32, (_G, _N), 0)
        zs = []
        for h_ref, b_ref in ((hm_ref, bm_ref), (hp_ref, bp_ref)):
            onehot = (gi == b_ref[...]).astype(jnp.float32)
            sums = jnp.dot(onehot, h_ref[...],
                           preferred_element_type=jnp.float32)
            cnt = jnp.sum(onehot, axis=1, keepdims=True)
            zs.append(sums / jnp.maximum(cnt, 1.0))
        z = jnp.concatenate(zs, axis=1)
        x1 = jnp.maximum(
            jnp.dot(z, w1[...], preferred_element_type=jnp.float32)
            + b1[...], 0.0)
        logits = (jnp.dot(x1, w2[...], preferred_element_type=jnp.float32)
                  + b2[...])
        o_ref[...] = 1.0 / (1.0 + jnp.exp(-logits))

    return pl.pallas_call(
        body,
        out_shape=jax.ShapeDtypeStruct((_G, 1), jnp.float32),
    )(hm, hp, bm.reshape(1, _N), bp.reshape(1, _N),
      fc1_W, fc1_b.reshape(1, _HID), fc2_W, fc2_b.reshape(1, 1))


def kernel(x_mol, edge_index_mol, edge_attr_mol, batch_mol, x_prot,
           edge_index_prot, edge_attr_prot, batch_prot, node_W_mol,
           node_b_mol, node_W_prot, node_b_prot, edge_W_mol, edge_b_mol,
           edge_W_prot, edge_b_prot, mol_W1, mol_b1, mol_W2, mol_b2, prot_W1,
           prot_b1, prot_W2, prot_b2, mp_WQ, mp_bQ, mp_WK, mp_bK, mp_WV,
           mp_bV, pm_WQ, pm_bQ, pm_WK, pm_bK, pm_WV, pm_bV, fc1_W, fc1_b,
           fc2_W, fc2_b):
    h_mol = _node_enc(x_mol, node_W_mol, node_b_mol)
    h_prot = _node_enc(x_prot, node_W_prot, node_b_prot)
    ea_mol = _edge_enc(edge_attr_mol, edge_W_mol, edge_b_mol)
    ea_prot = _edge_enc(edge_attr_prot, edge_W_prot, edge_b_prot)
    src_m, dst_m = edge_index_mol[0], edge_index_mol[1]
    src_p, dst_p = edge_index_prot[0], edge_index_prot[1]

    for i in range(3):
        aggs = _gine_agg(h_mol, ea_mol, src_m, dst_m,
                         h_prot, ea_prot, src_p, dst_p)
        h_mol = _gine_mlp(h_mol, aggs, 0, mol_W1[i], mol_b1[i],
                          mol_W2[i], mol_b2[i])
        h_prot = _gine_mlp(h_prot, aggs, 1, prot_W1[i], prot_b1[i],
                           prot_W2[i], prot_b2[i])

    q_m, kbd_p, vbd_p = _attn_prep(h_mol, h_prot, mp_WQ, mp_bQ, mp_WK, mp_bK,
                                   mp_WV, mp_bV)
    hm = _attn_apply(q_m, kbd_p, vbd_p, h_mol)
    q_p, kbd_m, vbd_m = _attn_prep(h_prot, h_mol, pm_WQ, pm_bQ, pm_WK, pm_bK,
                                   pm_WV, pm_bV)
    hp = _attn_apply(q_p, kbd_m, vbd_m, h_prot)

    out = _pool_head(hm, hp, batch_mol, batch_prot, fc1_W, fc1_b,
                     fc2_W, fc2_b)
    return out[:, 0]


# speedup vs baseline: 1.0692x; 1.0692x over previous
import functools

import jax
import jax.numpy as jnp
from jax import lax
from jax.experimental import pallas as pl
from jax.experimental.pallas import tpu as pltpu
from jax.experimental.pallas import tpu_sc as plsc

_HID = 64
_HEADS = 4
_HD = _HID // _HEADS
_G = 64
_N = 4096
_E = 65536

_NC = 2
_NS = 16
_EPW = _E // _NS
_CH = 128
_NCHUNK = _EPW // _CH
_RPS = _N // _NS


def _gine_agg(x_m, ea_m, src_m, dst_m, x_p, ea_p, src_p, dst_p):
    mesh = plsc.VectorSubcoreMesh(
        core_axis_name="c", subcore_axis_name="s",
        num_cores=_NC, num_subcores=_NS)

    @functools.partial(
        pl.kernel,
        mesh=mesh,
        compiler_params=pltpu.CompilerParams(use_tc_tiling_on_sc=False),
        out_type=jax.ShapeDtypeStruct((_NC, _N, _HID), jnp.float32),
        scratch_types=[
            pltpu.VMEM((4, _CH), jnp.int32),
            pltpu.VMEM((4, _CH), jnp.int32),
            pltpu.VMEM((4, _CH, _HID), jnp.float32),
            pltpu.VMEM((4, _CH, _HID), jnp.float32),
            pltpu.VMEM_SHARED((_N, _HID), jnp.float32),
            pltpu.SemaphoreType.DMA,
            pltpu.SemaphoreType.DMA,
            pltpu.SemaphoreType.DMA,
            pltpu.SemaphoreType.DMA,
            pltpu.SemaphoreType.DMA,
            pltpu.SemaphoreType.DMA,
            pltpu.SemaphoreType.DMA,
            pltpu.SemaphoreType.DMA,
        ],
    )
    def k(xm_hbm, eam_hbm, srcm_hbm, dstm_hbm, xp_hbm, eap_hbm, srcp_hbm,
          dstp_hbm, out_hbm, sidx, didx, rows, eabuf, acc, sem_in0, sem_in1,
          sem_in2, sem_in3, sem_g0, sem_g1, sem_g2, sem_g3):
        c = lax.axis_index("c")
        s = lax.axis_index("s")
        sem_in = (sem_in0, sem_in1, sem_in2, sem_in3)
        sem_g = (sem_g0, sem_g1, sem_g2, sem_g3)

        def zero_fill():
            @plsc.parallel_loop(0, _CH, step=1, unroll=8)
            def zero_row(i):
                for j in range(_HID // 16):
                    rows[0, i, pl.ds(j * 16, 16)] = jnp.zeros((16,),
                                                              jnp.float32)
            for r in range(_RPS // _CH):
                pltpu.sync_copy(rows.at[0],
                                acc.at[pl.ds(s * _RPS + r * _CH, _CH)])
            plsc.subcore_barrier()

        def run_graph(x_hbm, ea_hbm, src_hbm, dst_hbm, gidx, zero_fill):
            base = s * _EPW

            def start_in(t, b):
                e0 = base + t * _CH
                pltpu.async_copy(src_hbm.at[pl.ds(e0, _CH)], sidx.at[b],
                                 sem_in[b])
                pltpu.async_copy(dst_hbm.at[pl.ds(e0, _CH)], didx.at[b],
                                 sem_in[b])
                pltpu.async_copy(ea_hbm.at[pl.ds(e0, _CH)], eabuf.at[b],
                                 sem_in[b])

            def wait_in(b):
                pltpu.make_async_copy(src_hbm.at[pl.ds(0, _CH)], sidx.at[b],
                                      sem_in[b]).wait()
                pltpu.make_async_copy(dst_hbm.at[pl.ds(0, _CH)], didx.at[b],
                                      sem_in[b]).wait()
                pltpu.make_async_copy(ea_hbm.at[pl.ds(0, _CH)], eabuf.at[b],
                                      sem_in[b]).wait()

            def start_gather(b):
                pltpu.async_copy(x_hbm.at[sidx.at[b]], rows.at[b], sem_g[b])

            def wait_gather(b):
                pltpu.make_async_copy(x_hbm.at[sidx.at[b]], rows.at[b],
                                      sem_g[b]).wait()

            def compute(b):
                @plsc.parallel_loop(0, _CH, step=1, unroll=8)
                def body(i):
                    for j in range(_HID // 16):
                        sl = pl.ds(j * 16, 16)
                        rows[b, i, sl] = jnp.maximum(
                            rows[b, i, sl] + eabuf[b, i, sl], 0.0)

            def scatter(b):
                pltpu.sync_copy(rows.at[b], acc.at[didx.at[b]], add=True)

            for b in range(4):
                start_in(b, b)
            zero_fill()
            wait_in(0)
            start_gather(0)
            wait_in(1)
            start_gather(1)

            def step(t, b):
                b2 = (b + 2) % 4
                @pl.when(t + 2 < _NCHUNK)
                def _():
                    wait_in(b2)
                    start_gather(b2)
                wait_gather(b)
                compute(b)
                scatter(b)

                @pl.when(t + 4 < _NCHUNK)
                def _():
                    start_in(t + 4, b)

            def quad(p, carry):
                for b in range(4):
                    step(4 * p + b, b)
                return carry
            lax.fori_loop(0, _NCHUNK // 4, quad, 0)

            plsc.subcore_barrier()
            pltpu.sync_copy(acc.at[pl.ds(s * _RPS, _RPS)],
                            out_hbm.at[gidx, pl.ds(s * _RPS, _RPS)])

        @pl.when(c == 0)
        def _():
            run_graph(xm_hbm, eam_hbm, srcm_hbm, dstm_hbm, 0, zero_fill)

        @pl.when(c == 1)
        def _():
            run_graph(xp_hbm, eap_hbm, srcp_hbm, dstp_hbm, 1, zero_fill)

    return k(x_m, ea_m, src_m, dst_m, x_p, ea_p, src_p, dst_p)


def _node_enc(x, W, b):
    n, f = x.shape

    def body(x_ref, w_ref, b_ref, o_ref):
        o_ref[...] = jnp.dot(x_ref[...], w_ref[...],
                             preferred_element_type=jnp.float32) + b_ref[...]

    return pl.pallas_call(
        body,
        out_shape=jax.ShapeDtypeStruct((n, _HID), jnp.float32),
    )(x, W, b.reshape(1, _HID))


def _edge_enc(ea, W, b):
    e, f = ea.shape
    be = 16384

    def body(ea_ref, w_ref, b_ref, o_ref):
        o_ref[...] = jnp.dot(ea_ref[...], w_ref[...],
                             preferred_element_type=jnp.float32) + b_ref[...]

    return pl.pallas_call(
        body,
        grid=(e // be,),
        in_specs=[pl.BlockSpec((be, f), lambda i: (i, 0)),
                  pl.BlockSpec((f, _HID), lambda i: (0, 0)),
                  pl.BlockSpec((1, _HID), lambda i: (0, 0))],
        out_specs=pl.BlockSpec((be, _HID), lambda i: (i, 0)),
        out_shape=jax.ShapeDtypeStruct((e, _HID), jnp.float32),
    )(ea, W, b.reshape(1, _HID))


def _gine_mlp(x, aggs, g, W1, b1, W2, b2):
    def body(x_ref, a_ref, w1_ref, b1_ref, w2_ref, b2_ref, o_ref):
        h = x_ref[...] + a_ref[0]
        h1 = jnp.maximum(
            jnp.dot(h, w1_ref[...], preferred_element_type=jnp.float32)
            + b1_ref[...], 0.0)
        h2 = (jnp.dot(h1, w2_ref[...], preferred_element_type=jnp.float32)
              + b2_ref[...])
        o_ref[...] = jnp.maximum(h2, 0.0)

    return pl.pallas_call(
        body,
        grid=(1,),
        in_specs=[pl.BlockSpec((_N, _HID), lambda i: (0, 0)),
                  pl.BlockSpec((1, _N, _HID), lambda i, _g=g: (_g, 0, 0)),
                  pl.BlockSpec((_HID, _HID), lambda i: (0, 0)),
                  pl.BlockSpec((1, _HID), lambda i: (0, 0)),
                  pl.BlockSpec((_HID, _HID), lambda i: (0, 0)),
                  pl.BlockSpec((1, _HID), lambda i: (0, 0))],
        out_specs=pl.BlockSpec((_N, _HID), lambda i: (0, 0)),
        out_shape=jax.ShapeDtypeStruct((_N, _HID), jnp.float32),
    )(x, aggs, W1, b1.reshape(1, _HID), W2, b2.reshape(1, _HID))


def _attn_prep(qn, kn, WQ, bQ, WK, bK, WV, bV):
    def body(qn_ref, kn_ref, wq, bq, wk, bk, wv, bv, q_out, kbd_out, vbd_out):
        q = (jnp.dot(qn_ref[...], wq[...], preferred_element_type=jnp.float32)
             + bq[...])
        q_out[...] = (q * (1.0 / (_HD ** 0.5))).astype(jnp.bfloat16)
        k = (jnp.dot(kn_ref[...], wk[...], preferred_element_type=jnp.float32)
             + bk[...])
        kt = k.T
        drow = lax.broadcasted_iota(jnp.int32, (_HID, 1), 0) // _HD
        for h in range(_HEADS):
            kbd_out[:, h * _N:(h + 1) * _N] = (
                kt * (drow == h).astype(jnp.float32)).astype(jnp.bfloat16)
        v = (jnp.dot(kn_ref[...], wv[...], preferred_element_type=jnp.float32)
             + bv[...])
        dcol = lax.broadcasted_iota(jnp.int32, (1, _HID), 1) // _HD
        hcol = lax.broadcasted_iota(jnp.int32, (1, _HEADS), 1)
        for h in range(_HEADS):
            vbd_out[h * _N:(h + 1) * _N, :_HID] = (
                v * (dcol == h).astype(jnp.float32)).astype(jnp.bfloat16)
            vbd_out[h * _N:(h + 1) * _N, _HID:] = jnp.broadcast_to(
                (hcol == h).astype(jnp.bfloat16), (_N, _HEADS))

    return pl.pallas_call(
        body,
        out_shape=(jax.ShapeDtypeStruct((_N, _HID), jnp.bfloat16),
                   jax.ShapeDtypeStruct((_HID, _HEADS * _N), jnp.bfloat16),
                   jax.ShapeDtypeStruct((_HEADS * _N, _HID + _HEADS),
                                        jnp.bfloat16)),
    )(qn, kn, WQ, bQ.reshape(1, _HID), WK, bK.reshape(1, _HID),
      WV, bV.reshape(1, _HID))


_QB = 256


def _attn_apply(q_scaled, k_bd, v_bd, qn):
    def body(q_ref, kbd_ref, vbd_ref, qn_ref, o_ref):
        s = jnp.dot(q_ref[...], kbd_ref[...],
                    preferred_element_type=jnp.float32)
        ws = []
        for h in range(_HEADS):
            sh = s[:, h * _N:(h + 1) * _N]
            m = jnp.max(sh, axis=1, keepdims=True)
            ws.append(jnp.exp(sh - m).astype(jnp.bfloat16))
        w = jnp.concatenate(ws, axis=1)
        acc = jnp.dot(w, vbd_ref[...], preferred_element_type=jnp.float32)
        o = acc[:, :_HID]
        recip = 1.0 / acc[:, _HID:]
        rec64 = jnp.concatenate(
            [jnp.broadcast_to(recip[:, h:h + 1], (_QB, _HD))
             for h in range(_HEADS)], axis=1)
        o_ref[...] = qn_ref[...] + o * rec64

    return pl.pallas_call(
        body,
        grid=(_N // _QB,),
        in_specs=[pl.BlockSpec((_QB, _HID), lambda i: (i, 0)),
                  pl.BlockSpec((_HID, _HEADS * _N), lambda i: (0, 0)),
                  pl.BlockSpec((_HEADS * _N, _HID + _HEADS), lambda i: (0, 0)),
                  pl.BlockSpec((_QB, _HID), lambda i: (i, 0))],
        out_specs=pl.BlockSpec((_QB, _HID), lambda i: (i, 0)),
        out_shape=jax.ShapeDtypeStruct((_N, _HID), jnp.float32),
    )(q_scaled, k_bd, v_bd, qn)


def _pool_head(hm, hp, bm, bp, fc1_W, fc1_b, fc2_W, fc2_b):
    def body(hm_ref, hp_ref, bm_ref, bp_ref, w1, b1, w2, b2, o_ref):
        gi = lax.broadcasted_iota(jnp.int32, (_G, _N), 0)
        zs = []
        for h_ref, b_ref in ((hm_ref, bm_ref), (hp_ref, bp_ref)):
            onehot = (gi == b_ref[...]).astype(jnp.float32)
            sums = jnp.dot(onehot, h_ref[...],
                           preferred_element_type=jnp.float32)
            cnt = jnp.sum(onehot, axis=1, keepdims=True)
            zs.append(sums / jnp.maximum(cnt, 1.0))
        z = jnp.concatenate(zs, axis=1)
        x1 = jnp.maximum(
            jnp.dot(z, w1[...], preferred_element_type=jnp.float32)
            + b1[...], 0.0)
        logits = (jnp.dot(x1, w2[...], preferred_element_type=jnp.float32)
                  + b2[...])
        o_ref[...] = 1.0 / (1.0 + jnp.exp(-logits))

    return pl.pallas_call(
        body,
        out_shape=jax.ShapeDtypeStruct((_G, 1), jnp.float32),
    )(hm, hp, bm.reshape(1, _N), bp.reshape(1, _N),
      fc1_W, fc1_b.reshape(1, _HID), fc2_W, fc2_b.reshape(1, 1))


def kernel(x_mol, edge_index_mol, edge_attr_mol, batch_mol, x_prot,
           edge_index_prot, edge_attr_prot, batch_prot, node_W_mol,
           node_b_mol, node_W_prot, node_b_prot, edge_W_mol, edge_b_mol,
           edge_W_prot, edge_b_prot, mol_W1, mol_b1, mol_W2, mol_b2, prot_W1,
           prot_b1, prot_W2, prot_b2, mp_WQ, mp_bQ, mp_WK, mp_bK, mp_WV,
           mp_bV, pm_WQ, pm_bQ, pm_WK, pm_bK, pm_WV, pm_bV, fc1_W, fc1_b,
           fc2_W, fc2_b):
    h_mol = _node_enc(x_mol, node_W_mol, node_b_mol)
    h_prot = _node_enc(x_prot, node_W_prot, node_b_prot)
    ea_mol = _edge_enc(edge_attr_mol, edge_W_mol, edge_b_mol)
    ea_prot = _edge_enc(edge_attr_prot, edge_W_prot, edge_b_prot)
    src_m, dst_m = edge_index_mol[0], edge_index_mol[1]
    src_p, dst_p = edge_index_prot[0], edge_index_prot[1]

    for i in range(3):
        aggs = _gine_agg(h_mol, ea_mol, src_m, dst_m,
                         h_prot, ea_prot, src_p, dst_p)
        h_mol = _gine_mlp(h_mol, aggs, 0, mol_W1[i], mol_b1[i],
                          mol_W2[i], mol_b2[i])
        h_prot = _gine_mlp(h_prot, aggs, 1, prot_W1[i], prot_b1[i],
                           prot_W2[i], prot_b2[i])

    q_m, kbd_p, vbd_p = _attn_prep(h_mol, h_prot, mp_WQ, mp_bQ, mp_WK, mp_bK,
                                   mp_WV, mp_bV)
    hm = _attn_apply(q_m, kbd_p, vbd_p, h_mol)
    q_p, kbd_m, vbd_m = _attn_prep(h_prot, h_mol, pm_WQ, pm_bQ, pm_WK, pm_bK,
                                   pm_WV, pm_bV)
    hp = _attn_apply(q_p, kbd_m, vbd_m, h_prot)

    out = _pool_head(hm, hp, batch_mol, batch_prot, fc1_W, fc1_b,
                     fc2_W, fc2_b)
    return out[:, 0]


# speedup vs baseline: 1.0780x; 1.0083x over previous
import functools

import jax
import jax.numpy as jnp
from jax import lax
from jax.experimental import pallas as pl
from jax.experimental.pallas import tpu as pltpu
from jax.experimental.pallas import tpu_sc as plsc

_HID = 64
_HEADS = 4
_HD = _HID // _HEADS
_G = 64
_N = 4096
_E = 65536

_NC = 2
_NS = 16
_EPW = _E // _NS
_CH = 128
_NCHUNK = _EPW // _CH
_RPS = _N // _NS


def _gine_agg(x_m, ea_m, src_m, dst_m, x_p, ea_p, src_p, dst_p):
    mesh = plsc.VectorSubcoreMesh(
        core_axis_name="c", subcore_axis_name="s",
        num_cores=_NC, num_subcores=_NS)

    @functools.partial(
        pl.kernel,
        mesh=mesh,
        compiler_params=pltpu.CompilerParams(use_tc_tiling_on_sc=False),
        out_type=jax.ShapeDtypeStruct((_NC, _N, _HID), jnp.float32),
        scratch_types=[
            pltpu.VMEM((4, _CH), jnp.int32),
            pltpu.VMEM((4, _CH), jnp.int32),
            pltpu.VMEM((4, _CH, _HID), jnp.float32),
            pltpu.VMEM((4, _CH, _HID), jnp.float32),
            pltpu.VMEM_SHARED((_N, _HID), jnp.float32),
        ] + [pltpu.SemaphoreType.DMA] * 16,
    )
    def k(xm_hbm, eam_hbm, srcm_hbm, dstm_hbm, xp_hbm, eap_hbm, srcp_hbm,
          dstp_hbm, out_hbm, sidx, didx, rows, eabuf, acc, *sems):
        c = lax.axis_index("c")
        s = lax.axis_index("s")
        sem_in = sems[0:4]
        sem_di = sems[4:8]
        sem_g = sems[8:12]
        sem_sc = sems[12:16]

        def zero_fill():
            @plsc.parallel_loop(0, _CH, step=1, unroll=8)
            def zero_row(i):
                for j in range(_HID // 16):
                    rows[0, i, pl.ds(j * 16, 16)] = jnp.zeros((16,),
                                                              jnp.float32)
            for r in range(_RPS // _CH):
                pltpu.sync_copy(rows.at[0],
                                acc.at[pl.ds(s * _RPS + r * _CH, _CH)])
            plsc.subcore_barrier()

        def run_graph(x_hbm, ea_hbm, src_hbm, dst_hbm, gidx, zero_fill):
            base = s * _EPW

            def start_in(t, b):
                e0 = base + t * _CH
                pltpu.async_copy(src_hbm.at[pl.ds(e0, _CH)], sidx.at[b],
                                 sem_in[b])
                pltpu.async_copy(ea_hbm.at[pl.ds(e0, _CH)], eabuf.at[b],
                                 sem_in[b])

            def wait_in(b):
                pltpu.make_async_copy(src_hbm.at[pl.ds(0, _CH)], sidx.at[b],
                                      sem_in[b]).wait()
                pltpu.make_async_copy(ea_hbm.at[pl.ds(0, _CH)], eabuf.at[b],
                                      sem_in[b]).wait()

            def start_didx(t, b):
                e0 = base + t * _CH
                pltpu.async_copy(dst_hbm.at[pl.ds(e0, _CH)], didx.at[b],
                                 sem_di[b])

            def wait_didx(b):
                pltpu.make_async_copy(dst_hbm.at[pl.ds(0, _CH)], didx.at[b],
                                      sem_di[b]).wait()

            def start_gather(b):
                pltpu.async_copy(x_hbm.at[sidx.at[b]], rows.at[b], sem_g[b])

            def wait_gather(b):
                pltpu.make_async_copy(x_hbm.at[sidx.at[b]], rows.at[b],
                                      sem_g[b]).wait()

            def compute(b):
                @plsc.parallel_loop(0, _CH, step=1, unroll=8)
                def body(i):
                    for j in range(_HID // 16):
                        sl = pl.ds(j * 16, 16)
                        rows[b, i, sl] = jnp.maximum(
                            rows[b, i, sl] + eabuf[b, i, sl], 0.0)

            def start_scatter(b):
                pltpu.async_copy(rows.at[b], acc.at[didx.at[b]], sem_sc[b],
                                 add=True)

            def wait_scatter(b):
                pltpu.make_async_copy(rows.at[b], acc.at[didx.at[b]],
                                      sem_sc[b]).wait()

            for b in range(4):
                start_in(b, b)
            start_didx(0, 0)
            start_didx(1, 1)
            zero_fill()
            wait_in(0)
            start_gather(0)
            wait_in(1)
            start_gather(1)

            def step(t, b):
                b2 = (b + 2) % 4
                @pl.when(t + 2 < _NCHUNK)
                def _():
                    @pl.when(t >= 2)
                    def _():
                        wait_scatter(b2)
                    start_didx(t + 2, b2)
                    wait_in(b2)
                    start_gather(b2)
                wait_gather(b)
                compute(b)

                @pl.when(t + 4 < _NCHUNK)
                def _():
                    start_in(t + 4, b)
                wait_didx(b)
                start_scatter(b)

            def quad(p, carry):
                for b in range(4):
                    step(4 * p + b, b)
                return carry
            lax.fori_loop(0, _NCHUNK // 4, quad, 0)
            for b in range(4):
                wait_scatter(b)

            plsc.subcore_barrier()
            pltpu.sync_copy(acc.at[pl.ds(s * _RPS, _RPS)],
                            out_hbm.at[gidx, pl.ds(s * _RPS, _RPS)])

        @pl.when(c == 0)
        def _():
            run_graph(xm_hbm, eam_hbm, srcm_hbm, dstm_hbm, 0, zero_fill)

        @pl.when(c == 1)
        def _():
            run_graph(xp_hbm, eap_hbm, srcp_hbm, dstp_hbm, 1, zero_fill)

    return k(x_m, ea_m, src_m, dst_m, x_p, ea_p, src_p, dst_p)


def _node_enc(x, W, b):
    n, f = x.shape

    def body(x_ref, w_ref, b_ref, o_ref):
        o_ref[...] = jnp.dot(x_ref[...], w_ref[...],
                             preferred_element_type=jnp.float32) + b_ref[...]

    return pl.pallas_call(
        body,
        out_shape=jax.ShapeDtypeStruct((n, _HID), jnp.float32),
    )(x, W, b.reshape(1, _HID))


def _edge_enc(ea, W, b):
    e, f = ea.shape
    be = 16384

    def body(ea_ref, w_ref, b_ref, o_ref):
        o_ref[...] = jnp.dot(ea_ref[...], w_ref[...],
                             preferred_element_type=jnp.float32) + b_ref[...]

    return pl.pallas_call(
        body,
        grid=(e // be,),
        in_specs=[pl.BlockSpec((be, f), lambda i: (i, 0)),
                  pl.BlockSpec((f, _HID), lambda i: (0, 0)),
                  pl.BlockSpec((1, _HID), lambda i: (0, 0))],
        out_specs=pl.BlockSpec((be, _HID), lambda i: (i, 0)),
        out_shape=jax.ShapeDtypeStruct((e, _HID), jnp.float32),
    )(ea, W, b.reshape(1, _HID))


def _gine_mlp(x, aggs, g, W1, b1, W2, b2):
    def body(x_ref, a_ref, w1_ref, b1_ref, w2_ref, b2_ref, o_ref):
        h = x_ref[...] + a_ref[0]
        h1 = jnp.maximum(
            jnp.dot(h, w1_ref[...], preferred_element_type=jnp.float32)
            + b1_ref[...], 0.0)
        h2 = (jnp.dot(h1, w2_ref[...], preferred_element_type=jnp.float32)
              + b2_ref[...])
        o_ref[...] = jnp.maximum(h2, 0.0)

    return pl.pallas_call(
        body,
        grid=(1,),
        in_specs=[pl.BlockSpec((_N, _HID), lambda i: (0, 0)),
                  pl.BlockSpec((1, _N, _HID), lambda i, _g=g: (_g, 0, 0)),
                  pl.BlockSpec((_HID, _HID), lambda i: (0, 0)),
                  pl.BlockSpec((1, _HID), lambda i: (0, 0)),
                  pl.BlockSpec((_HID, _HID), lambda i: (0, 0)),
                  pl.BlockSpec((1, _HID), lambda i: (0, 0))],
        out_specs=pl.BlockSpec((_N, _HID), lambda i: (0, 0)),
        out_shape=jax.ShapeDtypeStruct((_N, _HID), jnp.float32),
    )(x, aggs, W1, b1.reshape(1, _HID), W2, b2.reshape(1, _HID))


def _attn_prep(qn, kn, WQ, bQ, WK, bK, WV, bV):
    def body(qn_ref, kn_ref, wq, bq, wk, bk, wv, bv, q_out, kbd_out, vbd_out):
        q = (jnp.dot(qn_ref[...], wq[...], preferred_element_type=jnp.float32)
             + bq[...])
        q_out[...] = (q * (1.0 / (_HD ** 0.5))).astype(jnp.bfloat16)
        k = (jnp.dot(kn_ref[...], wk[...], preferred_element_type=jnp.float32)
             + bk[...])
        kt = k.T
        drow = lax.broadcasted_iota(jnp.int32, (_HID, 1), 0) // _HD
        for h in range(_HEADS):
            kbd_out[:, h * _N:(h + 1) * _N] = (
                kt * (drow == h).astype(jnp.float32)).astype(jnp.bfloat16)
        v = (jnp.dot(kn_ref[...], wv[...], preferred_element_type=jnp.float32)
             + bv[...])
        dcol = lax.broadcasted_iota(jnp.int32, (1, _HID), 1) // _HD
        hcol = lax.broadcasted_iota(jnp.int32, (1, _HEADS), 1)
        for h in range(_HEADS):
            vbd_out[h * _N:(h + 1) * _N, :_HID] = (
                v * (dcol == h).astype(jnp.float32)).astype(jnp.bfloat16)
            vbd_out[h * _N:(h + 1) * _N, _HID:] = jnp.broadcast_to(
                (hcol == h).astype(jnp.bfloat16), (_N, _HEADS))

    return pl.pallas_call(
        body,
        out_shape=(jax.ShapeDtypeStruct((_N, _HID), jnp.bfloat16),
                   jax.ShapeDtypeStruct((_HID, _HEADS * _N), jnp.bfloat16),
                   jax.ShapeDtypeStruct((_HEADS * _N, _HID + _HEADS),
                                        jnp.bfloat16)),
    )(qn, kn, WQ, bQ.reshape(1, _HID), WK, bK.reshape(1, _HID),
      WV, bV.reshape(1, _HID))


_QB = 256


def _attn_apply(q_scaled, k_bd, v_bd, qn):
    def body(q_ref, kbd_ref, vbd_ref, qn_ref, o_ref):
        s = jnp.dot(q_ref[...], kbd_ref[...],
                    preferred_element_type=jnp.float32)
        ws = []
        for h in range(_HEADS):
            sh = s[:, h * _N:(h + 1) * _N]
            m = jnp.max(sh, axis=1, keepdims=True)
            ws.append(jnp.exp(sh - m).astype(jnp.bfloat16))
        w = jnp.concatenate(ws, axis=1)
        acc = jnp.dot(w, vbd_ref[...], preferred_element_type=jnp.float32)
        o = acc[:, :_HID]
        recip = 1.0 / acc[:, _HID:]
        rec64 = jnp.concatenate(
            [jnp.broadcast_to(recip[:, h:h + 1], (_QB, _HD))
             for h in range(_HEADS)], axis=1)
        o_ref[...] = qn_ref[...] + o * rec64

    return pl.pallas_call(
        body,
        grid=(_N // _QB,),
        in_specs=[pl.BlockSpec((_QB, _HID), lambda i: (i, 0)),
                  pl.BlockSpec((_HID, _HEADS * _N), lambda i: (0, 0)),
                  pl.BlockSpec((_HEADS * _N, _HID + _HEADS), lambda i: (0, 0)),
                  pl.BlockSpec((_QB, _HID), lambda i: (i, 0))],
        out_specs=pl.BlockSpec((_QB, _HID), lambda i: (i, 0)),
        out_shape=jax.ShapeDtypeStruct((_N, _HID), jnp.float32),
    )(q_scaled, k_bd, v_bd, qn)


def _pool_head(hm, hp, bm, bp, fc1_W, fc1_b, fc2_W, fc2_b):
    def body(hm_ref, hp_ref, bm_ref, bp_ref, w1, b1, w2, b2, o_ref):
        gi = lax.broadcasted_iota(jnp.int32, (_G, _N), 0)
        zs = []
        for h_ref, b_ref in ((hm_ref, bm_ref), (hp_ref, bp_ref)):
            onehot = (gi == b_ref[...]).astype(jnp.float32)
            sums = jnp.dot(onehot, h_ref[...],
                           preferred_element_type=jnp.float32)
            cnt = jnp.sum(onehot, axis=1, keepdims=True)
            zs.append(sums / jnp.maximum(cnt, 1.0))
        z = jnp.concatenate(zs, axis=1)
        x1 = jnp.maximum(
            jnp.dot(z, w1[...], preferred_element_type=jnp.float32)
            + b1[...], 0.0)
        logits = (jnp.dot(x1, w2[...], preferred_element_type=jnp.float32)
                  + b2[...])
        o_ref[...] = 1.0 / (1.0 + jnp.exp(-logits))

    return pl.pallas_call(
        body,
        out_shape=jax.ShapeDtypeStruct((_G, 1), jnp.float32),
    )(hm, hp, bm.reshape(1, _N), bp.reshape(1, _N),
      fc1_W, fc1_b.reshape(1, _HID), fc2_W, fc2_b.reshape(1, 1))


def kernel(x_mol, edge_index_mol, edge_attr_mol, batch_mol, x_prot,
           edge_index_prot, edge_attr_prot, batch_prot, node_W_mol,
           node_b_mol, node_W_prot, node_b_prot, edge_W_mol, edge_b_mol,
           edge_W_prot, edge_b_prot, mol_W1, mol_b1, mol_W2, mol_b2, prot_W1,
           prot_b1, prot_W2, prot_b2, mp_WQ, mp_bQ, mp_WK, mp_bK, mp_WV,
           mp_bV, pm_WQ, pm_bQ, pm_WK, pm_bK, pm_WV, pm_bV, fc1_W, fc1_b,
           fc2_W, fc2_b):
    h_mol = _node_enc(x_mol, node_W_mol, node_b_mol)
    h_prot = _node_enc(x_prot, node_W_prot, node_b_prot)
    ea_mol = _edge_enc(edge_attr_mol, edge_W_mol, edge_b_mol)
    ea_prot = _edge_enc(edge_attr_prot, edge_W_prot, edge_b_prot)
    src_m, dst_m = edge_index_mol[0], edge_index_mol[1]
    src_p, dst_p = edge_index_prot[0], edge_index_prot[1]

    for i in range(3):
        aggs = _gine_agg(h_mol, ea_mol, src_m, dst_m,
                         h_prot, ea_prot, src_p, dst_p)
        h_mol = _gine_mlp(h_mol, aggs, 0, mol_W1[i], mol_b1[i],
                          mol_W2[i], mol_b2[i])
        h_prot = _gine_mlp(h_prot, aggs, 1, prot_W1[i], prot_b1[i],
                           prot_W2[i], prot_b2[i])

    q_m, kbd_p, vbd_p = _attn_prep(h_mol, h_prot, mp_WQ, mp_bQ, mp_WK, mp_bK,
                                   mp_WV, mp_bV)
    hm = _attn_apply(q_m, kbd_p, vbd_p, h_mol)
    q_p, kbd_m, vbd_m = _attn_prep(h_prot, h_mol, pm_WQ, pm_bQ, pm_WK, pm_bK,
                                   pm_WV, pm_bV)
    hp = _attn_apply(q_p, kbd_m, vbd_m, h_prot)

    out = _pool_head(hm, hp, batch_mol, batch_prot, fc1_W, fc1_b,
                     fc2_W, fc2_b)
    return out[:, 0]
